# baseline (speedup 1.0000x reference)
import jax
import jax.numpy as jnp
from jax import lax
from jax.experimental import pallas as pl
from jax.experimental.pallas import tpu as pltpu
from jax.experimental.pallas import tpu_sc as plsc

N_NODES = 10000
NP = 10240
NR = NP // 128
N_EDGES = 320000
EP = 327680
ET = EP // 16
NCH = ET // 128
NPT = NP // 16
DQ = 32
D_IN = 128
D_HID = 256
D_OUT = 256
N_GRAPHS = 64
MB = 1280
GRID_M = NP // MB


def _k1_body(x_ref, w_ref, a2_ref, hcat_ref, asad_ref):
    j = pl.program_id(1)
    h = jnp.dot(x_ref[...], w_ref[...], preferred_element_type=jnp.float32)
    hcat_ref[...] = h
    part = jnp.dot(h, a2_ref[...], preferred_element_type=jnp.float32)

    @pl.when(j == 0)
    def _():
        asad_ref[...] = part

    @pl.when(j == 1)
    def _():
        asad_ref[...] += part


def _dense_layer1(x_pad, W1, A2):
    return pl.pallas_call(
        _k1_body,
        grid=(GRID_M, 2),
        in_specs=[
            pl.BlockSpec((MB, D_IN), lambda m, j: (m, 0)),
            pl.BlockSpec((D_IN, 128), lambda m, j: (0, j)),
            pl.BlockSpec((128, 128), lambda m, j: (j, 0)),
        ],
        out_specs=[
            pl.BlockSpec((MB, 128), lambda m, j: (j * GRID_M + m, 0)),
            pl.BlockSpec((MB, 128), lambda m, j: (m, 0)),
        ],
        out_shape=[
            jax.ShapeDtypeStruct((2 * NP, 128), jnp.float32),
            jax.ShapeDtypeStruct((NP, 128), jnp.float32),
        ],
    )(x_pad, W1, A2)


def _elu(v):
    return jnp.where(v > 0, v, jnp.exp(v) - 1.0)


def _k2_body(g0_ref, g1_ref, g2_ref, g3_ref, g4_ref, g5_ref, g6_ref, g7_ref,
             w_ref, a2_ref, hcat_ref, asad_ref):
    j = pl.program_id(1)
    h = None
    for i, g_ref in enumerate((g0_ref, g1_ref, g2_ref, g3_ref,
                               g4_ref, g5_ref, g6_ref, g7_ref)):
        part = jnp.dot(_elu(g_ref[...]), w_ref[i * DQ:(i + 1) * DQ, :],
                       preferred_element_type=jnp.float32)
        h = part if h is None else h + part
    hcat_ref[...] = h
    part = jnp.dot(h, a2_ref[...], preferred_element_type=jnp.float32)

    @pl.when(j == 0)
    def _():
        asad_ref[...] = part

    @pl.when(j == 1)
    def _():
        asad_ref[...] += part


def _dense_layer2(aqs, W2, A2):
    lo = lambda m, j: (m, 0)
    hi = lambda m, j: (GRID_M + m, 0)
    return pl.pallas_call(
        _k2_body,
        grid=(GRID_M, 2),
        in_specs=[
            pl.BlockSpec((MB, DQ), lo),
            pl.BlockSpec((MB, DQ), lo),
            pl.BlockSpec((MB, DQ), lo),
            pl.BlockSpec((MB, DQ), lo),
            pl.BlockSpec((MB, DQ), hi),
            pl.BlockSpec((MB, DQ), hi),
            pl.BlockSpec((MB, DQ), hi),
            pl.BlockSpec((MB, DQ), hi),
            pl.BlockSpec((D_HID, 128), lambda m, j: (0, j)),
            pl.BlockSpec((128, 128), lambda m, j: (j, 0)),
        ],
        out_specs=[
            pl.BlockSpec((MB, 128), lambda m, j: (j * GRID_M + m, 0)),
            pl.BlockSpec((MB, 128), lambda m, j: (m, 0)),
        ],
        out_shape=[
            jax.ShapeDtypeStruct((2 * NP, 128), jnp.float32),
            jax.ShapeDtypeStruct((NP, 128), jnp.float32),
        ],
    )(*aqs, *aqs, W2, A2)


def _k3_body(t0_ref, t1_ref, t2_ref, t3_ref, t4_ref, t5_ref, t6_ref, t7_ref,
             bat_ref, cx_ref, wc1_ref,
             bc1_ref, wc2_ref, bc2_ref, wo_ref, bo_ref, wf_ref, bf_ref,
             cp_ref, tp_ref, fp_ref):
    bat = bat_ref[...]
    gids = lax.broadcasted_iota(jnp.int32, (N_GRAPHS, NP), 0)
    onehot = (gids == bat[0:1, :]).astype(jnp.float32)
    cnt = jnp.sum(onehot, axis=1, keepdims=True)
    inv = 1.0 / jnp.maximum(cnt, 1.0)
    tq = [jnp.dot(onehot, t_ref[...], preferred_element_type=jnp.float32) * inv
          for t_ref in (t0_ref, t1_ref, t2_ref, t3_ref,
                        t4_ref, t5_ref, t6_ref, t7_ref)]
    z = jnp.maximum(
        jnp.dot(cx_ref[...], wc1_ref[...], preferred_element_type=jnp.float32)
        + bc1_ref[...], 0.0)
    code = (jnp.dot(z, wc2_ref[...], preferred_element_type=jnp.float32)
            + bc2_ref[...])

    def lsm(logits):
        m = jnp.max(logits, axis=-1, keepdims=True)
        sh = logits - m
        return sh - jnp.log(jnp.sum(jnp.exp(sh), axis=-1, keepdims=True))

    wo = wo_ref[...]
    bo = bo_ref[...]
    cp_ref[...] = lsm(jnp.dot(code, wo, preferred_element_type=jnp.float32) + bo)
    t_logit = bo
    for i in range(8):
        t_logit = t_logit + jnp.dot(tq[i], wo[i * DQ:(i + 1) * DQ, :],
                                    preferred_element_type=jnp.float32)
    tp_ref[...] = lsm(t_logit)
    wf = wf_ref[...]
    f_logit = (jnp.dot(code, wf[:256, :], preferred_element_type=jnp.float32)
               + bf_ref[...])
    for i in range(8):
        f_logit = f_logit + jnp.dot(
            tq[i], wf[256 + i * DQ:256 + (i + 1) * DQ, :],
            preferred_element_type=jnp.float32)
    fp_ref[...] = lsm(f_logit)


def _heads(aqs, bat2d, code_x, Wc1, bc1, Wc2, bc2, Wo, bo, Wf, bf):
    out128 = jax.ShapeDtypeStruct((N_GRAPHS, 128), jnp.float32)
    lo = lambda i: (0, 0)
    hi = lambda i: (1, 0)
    return pl.pallas_call(
        _k3_body,
        grid=(1,),
        in_specs=[
            pl.BlockSpec((NP, DQ), lo),
            pl.BlockSpec((NP, DQ), lo),
            pl.BlockSpec((NP, DQ), lo),
            pl.BlockSpec((NP, DQ), lo),
            pl.BlockSpec((NP, DQ), hi),
            pl.BlockSpec((NP, DQ), hi),
            pl.BlockSpec((NP, DQ), hi),
            pl.BlockSpec((NP, DQ), hi),
            pl.BlockSpec((8, NP), lambda i: (0, 0)),
            pl.BlockSpec((N_GRAPHS, 512), lambda i: (0, 0)),
            pl.BlockSpec((512, 256), lambda i: (0, 0)),
            pl.BlockSpec((1, 256), lambda i: (0, 0)),
            pl.BlockSpec((256, 256), lambda i: (0, 0)),
            pl.BlockSpec((1, 256), lambda i: (0, 0)),
            pl.BlockSpec((256, 128), lambda i: (0, 0)),
            pl.BlockSpec((1, 128), lambda i: (0, 0)),
            pl.BlockSpec((512, 128), lambda i: (0, 0)),
            pl.BlockSpec((1, 128), lambda i: (0, 0)),
        ],
        out_specs=[
            pl.BlockSpec((N_GRAPHS, 128), lambda i: (0, 0)),
            pl.BlockSpec((N_GRAPHS, 128), lambda i: (0, 0)),
            pl.BlockSpec((N_GRAPHS, 128), lambda i: (0, 0)),
        ],
        out_shape=[out128, out128, out128],
    )(*aqs, *aqs, bat2d, code_x, Wc1, bc1.reshape(1, -1),
      Wc2, bc2.reshape(1, -1), Wo, bo.reshape(1, -1), Wf, bf.reshape(1, -1))


def _edge_body(src_hbm, dst_hbm, a_s_hbm, a_d_hbm, hq_hbm,
               out0_hbm, out1_hbm, out2_hbm, out3_hbm,
               src_v, dst_v, a_s_v, a_d_v, denom_v, ee_v, gidx_v, rows_v,
               ridx_v, zb_v, acc_sh, denom_sh,
               semg0, semg1, semg2, semg3, sems0, sems1, sems2, sems3):
    c = lax.axis_index("c")
    s = lax.axis_index("s")

    pltpu.sync_copy(src_hbm.at[pl.ds(s * NCH, NCH)], src_v)
    pltpu.sync_copy(dst_hbm.at[pl.ds(s * NCH, NCH)], dst_v)
    pltpu.sync_copy(a_s_hbm, a_s_v)
    pltpu.sync_copy(a_d_hbm, a_d_v)

    zv = jnp.zeros((16,), jnp.float32)
    iota16 = lax.iota(jnp.int32, 16)

    def _z_denom(i, carry):
        for f in range(8):
            denom_v[i, pl.ds(f * 16, 16)] = zv
        return carry

    lax.fori_loop(0, NR, _z_denom, 0)

    for i in range(NR // 16):
        ridx_v[pl.ds(i * 16, 16)] = iota16 + (i * 16)
        for f in range(8):
            zb_v[i, pl.ds(f * 16, 16)] = zv

    def _z_rows(i, carry):
        for f in range(DQ // 16):
            rows_v[0, i, pl.ds(f * 16, 16)] = zv
        return carry

    lax.fori_loop(0, 128, _z_rows, 0)

    for i in range(NPT // 128):
        pltpu.sync_copy(rows_v.at[0], acc_sh.at[pl.ds(s * NPT + i * 128, 128)])
    pltpu.sync_copy(zb_v, denom_sh.at[pl.ds(s * (NR // 16), NR // 16)])
    plsc.subcore_barrier()

    def _p1(j, carry):
        for k in range(8):
            sl = pl.ds(k * 16, 16)
            sv = src_v[j, sl]
            dv = dst_v[j, sl]
            svr = lax.shift_right_logical(sv, 7)
            svc = lax.bitwise_and(sv, 127)
            dvr = lax.shift_right_logical(dv, 7)
            dvc = lax.bitwise_and(dv, 127)
            av = plsc.load_gather(a_s_v, [svr, svc])
            bv = plsc.load_gather(a_d_v, [dvr, dvc])
            e = av + bv
            e = jnp.where(e > 0, e, 0.2 * e)
            ee = jnp.exp(e)
            ee_v[j, sl] = ee
            plsc.addupdate_scatter(denom_v, [dvr, dvc], ee)
        return carry

    lax.fori_loop(0, NCH, _p1, 0)

    pltpu.sync_copy(denom_v, denom_sh.at[ridx_v], add=True)
    plsc.subcore_barrier()
    pltpu.sync_copy(denom_sh, denom_v)

    gsems = (semg0, semg1, semg2, semg3)
    ssems = (sems0, sems1, sems2, sems3)

    def _issue_gather(j, b, q_add):
        for k in range(8):
            sl = pl.ds(k * 16, 16)
            gidx_v[b, sl] = lax.shift_left(src_v[j, sl], 2) + q_add
        pltpu.async_copy(hq_hbm.at[gidx_v.at[b]], rows_v.at[b], gsems[b])

    def _wait_gather(b):
        pltpu.make_async_copy(hq_hbm.at[gidx_v.at[b]], rows_v.at[b],
                              gsems[b]).wait()

    def _wait_scatter(b):
        pltpu.make_async_copy(rows_v.at[b], acc_sh.at[dst_v.at[0]],
                              ssems[b]).wait()

    def _scale_scatter(j, b):
        def _scale(g, carry2):
            ev = ee_v[j, pl.ds(g * 16, 16)]
            e0 = g * 16
            for l in range(16):
                a = ev[l]
                for f in range(DQ // 16):
                    fs = pl.ds(f * 16, 16)
                    rows_v[b, e0 + l, fs] = rows_v[b, e0 + l, fs] * a
            return carry2

        lax.fori_loop(0, 8, _scale, 0)
        pltpu.async_copy(rows_v.at[b], acc_sh.at[dst_v.at[j]], ssems[b],
                         add=True)

    for qq in range(4):
        out_hbm = (out0_hbm, out1_hbm, out2_hbm, out3_hbm)[qq]
        q_add = 4 * c * NP + qq

        _issue_gather(0, 0, q_add)
        _issue_gather(1, 1, q_add)
        _issue_gather(2, 2, q_add)

        def _p2(tt, carry):
            j0 = 4 * tt
            for b in range(4):
                j = j0 + b
                bn = (b + 3) % 4

                @pl.when(j + 3 < NCH)
                def _():
                    @pl.when(j >= 1)
                    def _():
                        _wait_scatter(bn)
                    _issue_gather(j + 3, bn, q_add)

                _wait_gather(b)
                _scale_scatter(j, b)
            return carry

        lax.fori_loop(0, NCH // 4, _p2, 0)
        for b in range(4):
            _wait_scatter(b)

        def _z1(i, carry):
            for f in range(DQ // 16):
                rows_v[1, i, pl.ds(f * 16, 16)] = zv
            return carry

        lax.fori_loop(0, 128, _z1, 0)
        plsc.subcore_barrier()

        for i in range(NPT // 128):
            row0 = s * NPT + i * 128
            pltpu.sync_copy(acc_sh.at[pl.ds(row0, 128)], rows_v.at[0])
            r0 = lax.shift_right_logical(row0, 7)

            def _norm(g, carry):
                d = denom_v[r0, pl.ds(g * 16, 16)]
                rv = 1.0 / (d + 1e-16)
                n0 = g * 16
                for l in range(16):
                    r = rv[l]
                    for f in range(DQ // 16):
                        fs = pl.ds(f * 16, 16)
                        rows_v[0, n0 + l, fs] = rows_v[0, n0 + l, fs] * r
                return carry

            lax.fori_loop(0, 8, _norm, 0)
            pltpu.sync_copy(rows_v.at[0],
                            out_hbm.at[pl.ds(c * NP + row0, 128)])
            pltpu.sync_copy(rows_v.at[1], acc_sh.at[pl.ds(row0, 128)])
        plsc.subcore_barrier()


def _make_edge_kernel():
    f32 = jnp.float32
    mesh = plsc.VectorSubcoreMesh(core_axis_name="c", subcore_axis_name="s")
    return pl.kernel(
        _edge_body,
        out_type=[jax.ShapeDtypeStruct((2 * NP, DQ), f32)] * 4,
        mesh=mesh,
        compiler_params=pltpu.CompilerParams(needs_layout_passes=False,
                                             use_tc_tiling_on_sc=False),
        scratch_types=[
            pltpu.VMEM((NCH, 128), jnp.int32),
            pltpu.VMEM((NCH, 128), jnp.int32),
            pltpu.VMEM((NR, 128), f32),
            pltpu.VMEM((NR, 128), f32),
            pltpu.VMEM((NR, 128), f32),
            pltpu.VMEM((NCH, 128), f32),
            pltpu.VMEM((4, 128), jnp.int32),
            pltpu.VMEM((4, 128, DQ), f32),
            pltpu.VMEM((NR,), jnp.int32),
            pltpu.VMEM((NR // 16, 128), f32),
            pltpu.VMEM_SHARED((NP, DQ), f32),
            pltpu.VMEM_SHARED((NR, 128), f32),
            pltpu.SemaphoreType.DMA,
            pltpu.SemaphoreType.DMA,
            pltpu.SemaphoreType.DMA,
            pltpu.SemaphoreType.DMA,
            pltpu.SemaphoreType.DMA,
            pltpu.SemaphoreType.DMA,
            pltpu.SemaphoreType.DMA,
            pltpu.SemaphoreType.DMA,
        ],
    )


_EDGE_KERNEL = _make_edge_kernel()


def _edge_sc(src2d, dst2d, a_s2d, a_d2d, hq):
    return _EDGE_KERNEL(src2d, dst2d, a_s2d, a_d2d, hq)


def kernel(x, edge_index, batch, code_x, W1, a_src1, a_dst1, W2, a_src2,
           a_dst2, Wc1, bc1, Wc2, bc2, Wo, bo, Wf, bf):
    src = edge_index[0].astype(jnp.int32)
    dst = edge_index[1].astype(jnp.int32)
    epad = jnp.full((EP - N_EDGES,), NP - 1, jnp.int32)
    src2d = jnp.concatenate([src, epad]).reshape(16 * NCH, 128)
    dst2d = jnp.concatenate([dst, epad]).reshape(16 * NCH, 128)
    x_pad = jnp.pad(x, ((0, NP - N_NODES), (0, 0)))
    bat = jnp.pad(batch.astype(jnp.int32), (0, NP - N_NODES),
                  constant_values=N_GRAPHS)
    bat2d = jnp.broadcast_to(bat[None, :], (8, NP))

    def a2(a_src, a_dst):
        z = jnp.zeros((a_src.shape[0], 128), jnp.float32)
        return z.at[:, 0].set(a_src).at[:, 1].set(a_dst)

    hcat1, asad1 = _dense_layer1(x_pad, W1, a2(a_src1, a_dst1))
    aqs = _edge_sc(src2d, dst2d, asad1[:, 0].reshape(NR, 128),
                   asad1[:, 1].reshape(NR, 128),
                   hcat1.reshape(8 * NP, DQ))
    hcat2, asad2 = _dense_layer2(aqs, W2, a2(a_src2, a_dst2))
    bqs = _edge_sc(src2d, dst2d, asad2[:, 0].reshape(NR, 128),
                   asad2[:, 1].reshape(NR, 128),
                   hcat2.reshape(8 * NP, DQ))
    return _heads(bqs, bat2d, code_x, Wc1, bc1, Wc2, bc2, Wo, bo, Wf, bf)


# speedup vs baseline: 1.0362x; 1.0362x over previous
import jax
import jax.numpy as jnp
from jax import lax
from jax.experimental import pallas as pl
from jax.experimental.pallas import tpu as pltpu
from jax.experimental.pallas import tpu_sc as plsc

N_NODES = 10000
NP = 10240
NR = NP // 128
N_EDGES = 320000
EP = 327680
ET = EP // 16
NCH = ET // 128
NPT = NP // 16
DQ = 32
D_IN = 128
D_HID = 256
D_OUT = 256
N_GRAPHS = 64
MB = 1280
GRID_M = NP // MB


def _k1_body(x_ref, w_ref, a2_ref, hcat_ref, asad_ref):
    j = pl.program_id(1)
    h = jnp.dot(x_ref[...], w_ref[...], preferred_element_type=jnp.float32)
    hcat_ref[...] = h
    part = jnp.dot(h, a2_ref[...], preferred_element_type=jnp.float32)

    @pl.when(j == 0)
    def _():
        asad_ref[...] = part

    @pl.when(j == 1)
    def _():
        asad_ref[...] += part


def _dense_layer1(x_pad, W1, A2):
    return pl.pallas_call(
        _k1_body,
        grid=(GRID_M, 2),
        in_specs=[
            pl.BlockSpec((MB, D_IN), lambda m, j: (m, 0)),
            pl.BlockSpec((D_IN, 128), lambda m, j: (0, j)),
            pl.BlockSpec((128, 128), lambda m, j: (j, 0)),
        ],
        out_specs=[
            pl.BlockSpec((MB, 128), lambda m, j: (j * GRID_M + m, 0)),
            pl.BlockSpec((MB, 128), lambda m, j: (m, 0)),
        ],
        out_shape=[
            jax.ShapeDtypeStruct((2 * NP, 128), jnp.float32),
            jax.ShapeDtypeStruct((NP, 128), jnp.float32),
        ],
    )(x_pad, W1, A2)


def _elu(v):
    return jnp.where(v > 0, v, jnp.exp(v) - 1.0)


def _k2_body(g0_ref, g1_ref, g2_ref, g3_ref, g4_ref, g5_ref, g6_ref, g7_ref,
             w_ref, a2_ref, hcat_ref, asad_ref):
    j = pl.program_id(1)
    h = None
    for i, g_ref in enumerate((g0_ref, g1_ref, g2_ref, g3_ref,
                               g4_ref, g5_ref, g6_ref, g7_ref)):
        part = jnp.dot(_elu(g_ref[...]), w_ref[i * DQ:(i + 1) * DQ, :],
                       preferred_element_type=jnp.float32)
        h = part if h is None else h + part
    hcat_ref[...] = h
    part = jnp.dot(h, a2_ref[...], preferred_element_type=jnp.float32)

    @pl.when(j == 0)
    def _():
        asad_ref[...] = part

    @pl.when(j == 1)
    def _():
        asad_ref[...] += part


def _dense_layer2(aqs, W2, A2):
    lo = lambda m, j: (m, 0)
    hi = lambda m, j: (GRID_M + m, 0)
    return pl.pallas_call(
        _k2_body,
        grid=(GRID_M, 2),
        in_specs=[
            pl.BlockSpec((MB, DQ), lo),
            pl.BlockSpec((MB, DQ), lo),
            pl.BlockSpec((MB, DQ), lo),
            pl.BlockSpec((MB, DQ), lo),
            pl.BlockSpec((MB, DQ), hi),
            pl.BlockSpec((MB, DQ), hi),
            pl.BlockSpec((MB, DQ), hi),
            pl.BlockSpec((MB, DQ), hi),
            pl.BlockSpec((D_HID, 128), lambda m, j: (0, j)),
            pl.BlockSpec((128, 128), lambda m, j: (j, 0)),
        ],
        out_specs=[
            pl.BlockSpec((MB, 128), lambda m, j: (j * GRID_M + m, 0)),
            pl.BlockSpec((MB, 128), lambda m, j: (m, 0)),
        ],
        out_shape=[
            jax.ShapeDtypeStruct((2 * NP, 128), jnp.float32),
            jax.ShapeDtypeStruct((NP, 128), jnp.float32),
        ],
    )(*aqs, *aqs, W2, A2)


def _k3_body(t0_ref, t1_ref, t2_ref, t3_ref, t4_ref, t5_ref, t6_ref, t7_ref,
             bat_ref, cx_ref, wc1_ref,
             bc1_ref, wc2_ref, bc2_ref, wo_ref, bo_ref, wf_ref, bf_ref,
             cp_ref, tp_ref, fp_ref):
    bat = bat_ref[...]
    gids = lax.broadcasted_iota(jnp.int32, (N_GRAPHS, NP), 0)
    onehot = (gids == bat[0:1, :]).astype(jnp.float32)
    cnt = jnp.sum(onehot, axis=1, keepdims=True)
    inv = 1.0 / jnp.maximum(cnt, 1.0)
    tq = [jnp.dot(onehot, t_ref[...], preferred_element_type=jnp.float32) * inv
          for t_ref in (t0_ref, t1_ref, t2_ref, t3_ref,
                        t4_ref, t5_ref, t6_ref, t7_ref)]
    z = jnp.maximum(
        jnp.dot(cx_ref[...], wc1_ref[...], preferred_element_type=jnp.float32)
        + bc1_ref[...], 0.0)
    code = (jnp.dot(z, wc2_ref[...], preferred_element_type=jnp.float32)
            + bc2_ref[...])

    def lsm(logits):
        m = jnp.max(logits, axis=-1, keepdims=True)
        sh = logits - m
        return sh - jnp.log(jnp.sum(jnp.exp(sh), axis=-1, keepdims=True))

    wo = wo_ref[...]
    bo = bo_ref[...]
    cp_ref[...] = lsm(jnp.dot(code, wo, preferred_element_type=jnp.float32) + bo)
    t_logit = bo
    for i in range(8):
        t_logit = t_logit + jnp.dot(tq[i], wo[i * DQ:(i + 1) * DQ, :],
                                    preferred_element_type=jnp.float32)
    tp_ref[...] = lsm(t_logit)
    wf = wf_ref[...]
    f_logit = (jnp.dot(code, wf[:256, :], preferred_element_type=jnp.float32)
               + bf_ref[...])
    for i in range(8):
        f_logit = f_logit + jnp.dot(
            tq[i], wf[256 + i * DQ:256 + (i + 1) * DQ, :],
            preferred_element_type=jnp.float32)
    fp_ref[...] = lsm(f_logit)


def _heads(aqs, bat2d, code_x, Wc1, bc1, Wc2, bc2, Wo, bo, Wf, bf):
    out128 = jax.ShapeDtypeStruct((N_GRAPHS, 128), jnp.float32)
    lo = lambda i: (0, 0)
    hi = lambda i: (1, 0)
    return pl.pallas_call(
        _k3_body,
        grid=(1,),
        in_specs=[
            pl.BlockSpec((NP, DQ), lo),
            pl.BlockSpec((NP, DQ), lo),
            pl.BlockSpec((NP, DQ), lo),
            pl.BlockSpec((NP, DQ), lo),
            pl.BlockSpec((NP, DQ), hi),
            pl.BlockSpec((NP, DQ), hi),
            pl.BlockSpec((NP, DQ), hi),
            pl.BlockSpec((NP, DQ), hi),
            pl.BlockSpec((8, NP), lambda i: (0, 0)),
            pl.BlockSpec((N_GRAPHS, 512), lambda i: (0, 0)),
            pl.BlockSpec((512, 256), lambda i: (0, 0)),
            pl.BlockSpec((1, 256), lambda i: (0, 0)),
            pl.BlockSpec((256, 256), lambda i: (0, 0)),
            pl.BlockSpec((1, 256), lambda i: (0, 0)),
            pl.BlockSpec((256, 128), lambda i: (0, 0)),
            pl.BlockSpec((1, 128), lambda i: (0, 0)),
            pl.BlockSpec((512, 128), lambda i: (0, 0)),
            pl.BlockSpec((1, 128), lambda i: (0, 0)),
        ],
        out_specs=[
            pl.BlockSpec((N_GRAPHS, 128), lambda i: (0, 0)),
            pl.BlockSpec((N_GRAPHS, 128), lambda i: (0, 0)),
            pl.BlockSpec((N_GRAPHS, 128), lambda i: (0, 0)),
        ],
        out_shape=[out128, out128, out128],
    )(*aqs, *aqs, bat2d, code_x, Wc1, bc1.reshape(1, -1),
      Wc2, bc2.reshape(1, -1), Wo, bo.reshape(1, -1), Wf, bf.reshape(1, -1))


def _edge_body(src_hbm, dst_hbm, a_s_hbm, a_d_hbm, hq_hbm,
               out0_hbm, out1_hbm, out2_hbm, out3_hbm,
               src_v, dst_v, a_s_v, a_d_v, denom_v, ee_v, gidx_v, rows_v,
               ridx_v, zb_v, acc_sh, denom_sh,
               semg0, semg1, semg2, semg3, sems0, sems1, sems2, sems3):
    c = lax.axis_index("c")
    s = lax.axis_index("s")

    pltpu.sync_copy(src_hbm.at[pl.ds(s * NCH, NCH)], src_v)
    pltpu.sync_copy(dst_hbm.at[pl.ds(s * NCH, NCH)], dst_v)
    pltpu.sync_copy(a_s_hbm, a_s_v)
    pltpu.sync_copy(a_d_hbm, a_d_v)

    zv = jnp.zeros((16,), jnp.float32)
    iota16 = lax.iota(jnp.int32, 16)

    def _z_denom(i, carry):
        for f in range(8):
            denom_v[i, pl.ds(f * 16, 16)] = zv
        return carry

    lax.fori_loop(0, NR, _z_denom, 0)

    for i in range(NR // 16):
        ridx_v[pl.ds(i * 16, 16)] = iota16 + (i * 16)
        for f in range(8):
            zb_v[i, pl.ds(f * 16, 16)] = zv

    def _z_rows(i, carry):
        for f in range(DQ // 16):
            rows_v[0, i, pl.ds(f * 16, 16)] = zv
        return carry

    lax.fori_loop(0, 128, _z_rows, 0)

    for i in range(NPT // 128):
        pltpu.sync_copy(rows_v.at[0], acc_sh.at[pl.ds(s * NPT + i * 128, 128)])
    pltpu.sync_copy(zb_v, denom_sh.at[pl.ds(s * (NR // 16), NR // 16)])
    plsc.subcore_barrier()

    def _p1(j, carry):
        for k in range(8):
            sl = pl.ds(k * 16, 16)
            sv = src_v[j, sl]
            dv = dst_v[j, sl]
            svr = lax.shift_right_logical(sv, 7)
            svc = lax.bitwise_and(sv, 127)
            dvr = lax.shift_right_logical(dv, 7)
            dvc = lax.bitwise_and(dv, 127)
            av = plsc.load_gather(a_s_v, [svr, svc])
            bv = plsc.load_gather(a_d_v, [dvr, dvc])
            e = av + bv
            e = jnp.where(e > 0, e, 0.2 * e)
            ee = jnp.exp(e)
            ee_v[j, sl] = ee
            plsc.addupdate_scatter(denom_v, [dvr, dvc], ee)
        return carry

    lax.fori_loop(0, NCH, _p1, 0)

    pltpu.sync_copy(denom_v, denom_sh.at[ridx_v], add=True)
    plsc.subcore_barrier()
    pltpu.sync_copy(denom_sh, denom_v)

    gsems = (semg0, semg1, semg2, semg3)
    ssems = (sems0, sems1, sems2, sems3)

    def _issue_gather(j, b, q_add):
        for k in range(8):
            sl = pl.ds(k * 16, 16)
            gidx_v[b, sl] = lax.shift_left(src_v[j, sl], 2) + q_add
        pltpu.async_copy(hq_hbm.at[gidx_v.at[b]], rows_v.at[b], gsems[b])

    def _wait_gather(b):
        pltpu.make_async_copy(hq_hbm.at[gidx_v.at[b]], rows_v.at[b],
                              gsems[b]).wait()

    def _wait_scatter(b):
        pltpu.make_async_copy(rows_v.at[b], acc_sh.at[dst_v.at[0]],
                              ssems[b]).wait()

    def _scale_scatter(j, b):
        def _scale(g, carry2):
            ev = ee_v[j, pl.ds(g * 16, 16)]
            e0 = g * 16
            for l in range(16):
                a = ev[l]
                for f in range(DQ // 16):
                    fs = pl.ds(f * 16, 16)
                    rows_v[b, e0 + l, fs] = rows_v[b, e0 + l, fs] * a
            return carry2

        lax.fori_loop(0, 8, _scale, 0)
        pltpu.async_copy(rows_v.at[b], acc_sh.at[dst_v.at[j]], ssems[b],
                         add=True)

    for qq in range(4):
        out_hbm = (out0_hbm, out1_hbm, out2_hbm, out3_hbm)[qq]
        q_add = 4 * c * NP + qq

        _issue_gather(0, 0, q_add)
        _issue_gather(1, 1, q_add)

        def _p2(tt, carry):
            j0 = 4 * tt
            for b in range(4):
                j = j0 + b
                bn = (b + 2) % 4

                @pl.when(j + 2 < NCH)
                def _():
                    @pl.when(j >= 2)
                    def _():
                        _wait_scatter(bn)
                    _issue_gather(j + 2, bn, q_add)

                _wait_gather(b)
                _scale_scatter(j, b)
            return carry

        lax.fori_loop(0, NCH // 4, _p2, 0)
        for b in range(4):
            _wait_scatter(b)

        def _z1(i, carry):
            for f in range(DQ // 16):
                rows_v[1, i, pl.ds(f * 16, 16)] = zv
            return carry

        lax.fori_loop(0, 128, _z1, 0)
        plsc.subcore_barrier()

        NSL = NPT // 128
        bufs = tuple((0, 2)[i % 2] for i in range(NSL))

        def _slice_in(i):
            row0 = s * NPT + i * 128
            pltpu.async_copy(acc_sh.at[pl.ds(row0, 128)],
                             rows_v.at[bufs[i]], gsems[bufs[i]])

        def _wait_in(i):
            row0 = s * NPT + i * 128
            pltpu.make_async_copy(acc_sh.at[pl.ds(row0, 128)],
                                  rows_v.at[bufs[i]], gsems[bufs[i]]).wait()

        _slice_in(0)
        for i in range(NSL):
            b = bufs[i]
            row0 = s * NPT + i * 128
            if i + 1 < NSL:
                _slice_in(i + 1)
            _wait_in(i)
            r0 = lax.shift_right_logical(row0, 7)

            def _norm(g, carry):
                d = denom_v[r0, pl.ds(g * 16, 16)]
                rv = 1.0 / (d + 1e-16)
                n0 = g * 16
                for l in range(16):
                    r = rv[l]
                    for f in range(DQ // 16):
                        fs = pl.ds(f * 16, 16)
                        rows_v[b, n0 + l, fs] = rows_v[b, n0 + l, fs] * r
                return carry

            lax.fori_loop(0, 8, _norm, 0)
            pltpu.async_copy(rows_v.at[b],
                             out_hbm.at[pl.ds(c * NP + row0, 128)], ssems[b])
            if i + 2 < NSL:
                pltpu.make_async_copy(
                    rows_v.at[b],
                    out_hbm.at[pl.ds(c * NP + row0, 128)], ssems[b]).wait()
            if qq < 3:
                pltpu.sync_copy(rows_v.at[1], acc_sh.at[pl.ds(row0, 128)])
        for i in (NSL - 2, NSL - 1):
            b = bufs[i]
            row0 = s * NPT + i * 128
            pltpu.make_async_copy(
                rows_v.at[b],
                out_hbm.at[pl.ds(c * NP + row0, 128)], ssems[b]).wait()
        plsc.subcore_barrier()


def _make_edge_kernel():
    f32 = jnp.float32
    mesh = plsc.VectorSubcoreMesh(core_axis_name="c", subcore_axis_name="s")
    return pl.kernel(
        _edge_body,
        out_type=[jax.ShapeDtypeStruct((2 * NP, DQ), f32)] * 4,
        mesh=mesh,
        compiler_params=pltpu.CompilerParams(needs_layout_passes=False,
                                             use_tc_tiling_on_sc=False),
        scratch_types=[
            pltpu.VMEM((NCH, 128), jnp.int32),
            pltpu.VMEM((NCH, 128), jnp.int32),
            pltpu.VMEM((NR, 128), f32),
            pltpu.VMEM((NR, 128), f32),
            pltpu.VMEM((NR, 128), f32),
            pltpu.VMEM((NCH, 128), f32),
            pltpu.VMEM((4, 128), jnp.int32),
            pltpu.VMEM((4, 128, DQ), f32),
            pltpu.VMEM((NR,), jnp.int32),
            pltpu.VMEM((NR // 16, 128), f32),
            pltpu.VMEM_SHARED((NP, DQ), f32),
            pltpu.VMEM_SHARED((NR, 128), f32),
            pltpu.SemaphoreType.DMA,
            pltpu.SemaphoreType.DMA,
            pltpu.SemaphoreType.DMA,
            pltpu.SemaphoreType.DMA,
            pltpu.SemaphoreType.DMA,
            pltpu.SemaphoreType.DMA,
            pltpu.SemaphoreType.DMA,
            pltpu.SemaphoreType.DMA,
        ],
    )


_EDGE_KERNEL = _make_edge_kernel()


def _edge_sc(src2d, dst2d, a_s2d, a_d2d, hq):
    return _EDGE_KERNEL(src2d, dst2d, a_s2d, a_d2d, hq)


def kernel(x, edge_index, batch, code_x, W1, a_src1, a_dst1, W2, a_src2,
           a_dst2, Wc1, bc1, Wc2, bc2, Wo, bo, Wf, bf):
    src = edge_index[0].astype(jnp.int32)
    dst = edge_index[1].astype(jnp.int32)
    epad = jnp.full((EP - N_EDGES,), NP - 1, jnp.int32)
    src2d = jnp.concatenate([src, epad]).reshape(16 * NCH, 128)
    dst2d = jnp.concatenate([dst, epad]).reshape(16 * NCH, 128)
    x_pad = jnp.pad(x, ((0, NP - N_NODES), (0, 0)))
    bat = jnp.pad(batch.astype(jnp.int32), (0, NP - N_NODES),
                  constant_values=N_GRAPHS)
    bat2d = jnp.broadcast_to(bat[None, :], (8, NP))

    def a2(a_src, a_dst):
        z = jnp.zeros((a_src.shape[0], 128), jnp.float32)
        return z.at[:, 0].set(a_src).at[:, 1].set(a_dst)

    hcat1, asad1 = _dense_layer1(x_pad, W1, a2(a_src1, a_dst1))
    aqs = _edge_sc(src2d, dst2d, asad1[:, 0].reshape(NR, 128),
                   asad1[:, 1].reshape(NR, 128),
                   hcat1.reshape(8 * NP, DQ))
    hcat2, asad2 = _dense_layer2(aqs, W2, a2(a_src2, a_dst2))
    bqs = _edge_sc(src2d, dst2d, asad2[:, 0].reshape(NR, 128),
                   asad2[:, 1].reshape(NR, 128),
                   hcat2.reshape(8 * NP, DQ))
    return _heads(bqs, bat2d, code_x, Wc1, bc1, Wc2, bc2, Wo, bo, Wf, bf)
